# Initial kernel scaffold; baseline (speedup 1.0000x reference)
import jax
import jax.numpy as jnp
from jax.experimental import pallas as pl


def kernel(x, edge_index, W1, b1, g1, be1, rm1, rv1, W2, b2, g2, be2, rm2, rv2, W3, b3):
    raise NotImplementedError("write your pallas kernel here")


# speedup vs baseline: 7.7082x; 7.7082x over previous
import functools

import jax
import jax.numpy as jnp
from jax import lax
from jax.experimental import pallas as pl
from jax.experimental.pallas import tpu as pltpu
from jax.experimental.pallas import tpu_sc as plsc

N = 10000
F_IN = 128
H = 256
C = 40
HHALF = H // 2

NC = 2
NS = 16
K = 128

BT = 128
NT = 79 * BT
RPT = NT // NS

_MESH = plsc.VectorSubcoreMesh(
    core_axis_name="c", subcore_axis_name="s", num_cores=NC, num_subcores=NS)


def _deg_body(didx, zeros, ones, out, di_v, ones_v, s1, s2, s3, acc):
    c = lax.axis_index("c")
    s = lax.axis_index("s")
    wid = c * NS + s
    nb = didx.shape[1]
    pltpu.async_copy(ones, ones_v, s1).wait()
    pltpu.async_copy(zeros.at[pl.ds(s * RPT, RPT)],
                     acc.at[pl.ds(s * RPT, RPT)], s2).wait()
    plsc.subcore_barrier()

    def body(b, carry):
        pltpu.async_copy(didx.at[wid, b], di_v, s1).wait()
        for j in range(K // 16):
            iv = di_v[pl.ds(j * 16, 16)]
            pltpu.async_copy(ones_v, acc.at[iv], s3, add=True).wait()
        return carry

    lax.fori_loop(0, nb, body, 0)
    plsc.subcore_barrier()
    pltpu.async_copy(acc.at[pl.ds(s * RPT, RPT)],
                     out.at[c, pl.ds(s * RPT, RPT)], s2).wait()


def _agg_body(table, sidx, didx, out, si_v, di_v, buf, s1, s2, s3, s4, acc):
    c = lax.axis_index("c")
    s = lax.axis_index("s")
    nb = sidx.shape[2]
    pltpu.async_copy(table.at[pl.ds(c * NT + s * RPT, RPT)],
                     acc.at[pl.ds(s * RPT, RPT)], s1).wait()
    plsc.subcore_barrier()

    def body(b, carry):
        pltpu.async_copy(sidx.at[c, s, b], si_v, s2).wait()
        pltpu.async_copy(didx.at[s, b], di_v, s2).wait()
        pltpu.async_copy(table.at[si_v], buf, s3).wait()
        for j in range(K // 16):
            iv = di_v[pl.ds(j * 16, 16)]
            pltpu.async_copy(buf.at[pl.ds(j * 16, 16)], acc.at[iv], s4,
                             add=True).wait()
        return carry

    lax.fori_loop(0, nb, body, 0)
    plsc.subcore_barrier()
    pltpu.async_copy(acc.at[pl.ds(s * RPT, RPT)],
                     out.at[c, pl.ds(s * RPT, RPT)], s1).wait()


def _make_deg_call(nbd):
    return pl.kernel(
        _deg_body,
        out_type=jax.ShapeDtypeStruct((NC, NT, HHALF), jnp.float32),
        mesh=_MESH,
        scratch_types=[
            pltpu.VMEM((K,), jnp.int32),
            pltpu.VMEM((16, HHALF), jnp.float32),
            pltpu.SemaphoreType.DMA,
            pltpu.SemaphoreType.DMA,
            pltpu.SemaphoreType.DMA,
            pltpu.VMEM_SHARED((NT, HHALF), jnp.float32),
        ],
    )


def _make_agg_call():
    return pl.kernel(
        _agg_body,
        out_type=jax.ShapeDtypeStruct((NC, NT, HHALF), jnp.float32),
        mesh=_MESH,
        scratch_types=[
            pltpu.VMEM((K,), jnp.int32),
            pltpu.VMEM((K,), jnp.int32),
            pltpu.VMEM((K, HHALF), jnp.float32),
            pltpu.SemaphoreType.DMA,
            pltpu.SemaphoreType.DMA,
            pltpu.SemaphoreType.DMA,
            pltpu.SemaphoreType.DMA,
            pltpu.VMEM_SHARED((NT, HHALF), jnp.float32),
        ],
    )


def _prep_body(x_ref, w1_ref, deg_ref, hs_ref, dinv_ref):
    deg = deg_ref[0] + deg_ref[1]
    dinv = lax.rsqrt(deg[:, :1] + 1.0)
    h = jnp.dot(x_ref[...], w1_ref[...], preferred_element_type=jnp.float32)
    hs = h * dinv
    hs_ref[0] = hs[:, :HHALF]
    hs_ref[1] = hs[:, HHALF:]
    dinv_ref[...] = dinv * jnp.ones((1, 16), jnp.float32)


def _bn_relu(S_ref, dinv_ref, b_ref, g_ref, be_ref, rm_ref, rv_ref):
    dinv = dinv_ref[:, :1]
    agg = jnp.concatenate([S_ref[0], S_ref[1]], axis=1) * dinv + b_ref[...]
    scale = g_ref[...] * lax.rsqrt(rv_ref[...] + 1e-5)
    v = (agg - rm_ref[...]) * scale + be_ref[...]
    return jnp.maximum(v, 0.0), dinv


def _mid_body(S_ref, dinv_ref, w2_ref, b_ref, g_ref, be_ref, rm_ref, rv_ref,
              hs_ref):
    v, dinv = _bn_relu(S_ref, dinv_ref, b_ref, g_ref, be_ref, rm_ref, rv_ref)
    h2 = jnp.dot(v, w2_ref[...], preferred_element_type=jnp.float32)
    hs = h2 * dinv
    hs_ref[0] = hs[:, :HHALF]
    hs_ref[1] = hs[:, HHALF:]


def _fin_body(S_ref, dinv_ref, w3_ref, b_ref, g_ref, be_ref, rm_ref, rv_ref,
              b3_ref, out_ref):
    v, _ = _bn_relu(S_ref, dinv_ref, b_ref, g_ref, be_ref, rm_ref, rv_ref)
    out_ref[...] = (jnp.dot(v, w3_ref[...], preferred_element_type=jnp.float32)
                    + b3_ref[...])


_GRID = NT // BT

_row_spec = pl.BlockSpec((BT, F_IN), lambda i: (i, 0))
_S_spec = pl.BlockSpec((NC, BT, HHALF), lambda i: (0, i, 0))
_deg_spec = pl.BlockSpec((NC, BT, HHALF), lambda i: (0, i, 0))
_dinv_spec = pl.BlockSpec((BT, 16), lambda i: (i, 0))
_vecH_spec = pl.BlockSpec((1, H), lambda i: (0, 0))


def _prep_call(x, W1, deg2):
    return pl.pallas_call(
        _prep_body,
        grid=(_GRID,),
        in_specs=[_row_spec,
                  pl.BlockSpec((F_IN, H), lambda i: (0, 0)),
                  _deg_spec],
        out_specs=[pl.BlockSpec((NC, BT, HHALF), lambda i: (0, i, 0)),
                   _dinv_spec],
        out_shape=[jax.ShapeDtypeStruct((NC, NT, HHALF), jnp.float32),
                   jax.ShapeDtypeStruct((NT, 16), jnp.float32)],
    )(x, W1, deg2)


def _mid_call(S1, dinv, W2, b, g, be, rm, rv):
    return pl.pallas_call(
        _mid_body,
        grid=(_GRID,),
        in_specs=[_S_spec, _dinv_spec,
                  pl.BlockSpec((H, H), lambda i: (0, 0)),
                  _vecH_spec, _vecH_spec, _vecH_spec, _vecH_spec, _vecH_spec],
        out_specs=pl.BlockSpec((NC, BT, HHALF), lambda i: (0, i, 0)),
        out_shape=jax.ShapeDtypeStruct((NC, NT, HHALF), jnp.float32),
    )(S1, dinv, W2, b, g, be, rm, rv)


def _fin_call(S2, dinv, W3, b, g, be, rm, rv, b3):
    return pl.pallas_call(
        _fin_body,
        grid=(_GRID,),
        in_specs=[_S_spec, _dinv_spec,
                  pl.BlockSpec((H, C), lambda i: (0, 0)),
                  _vecH_spec, _vecH_spec, _vecH_spec, _vecH_spec, _vecH_spec,
                  pl.BlockSpec((1, C), lambda i: (0, 0))],
        out_specs=pl.BlockSpec((BT, C), lambda i: (i, 0)),
        out_shape=jax.ShapeDtypeStruct((N, C), jnp.float32),
    )(S2, dinv, W3, b, g, be, rm, rv, b3)


def _ceil_to(a, m):
    return -(-a // m) * m


def kernel(x, edge_index, W1, b1, g1, be1, rm1, rv1, W2, b2, g2, be2, rm2,
           rv2, W3, b3):
    src = edge_index[0].astype(jnp.int32)
    dst = edge_index[1].astype(jnp.int32)
    E = src.shape[0]

    epd = _ceil_to(E, NC * NS * K)
    nbd = epd // (NC * NS * K)
    didx_deg = jnp.concatenate(
        [dst, jnp.full((epd - E,), N, jnp.int32)]).reshape(NC * NS, nbd, K)

    epa = _ceil_to(E, NS * K)
    nba = epa // (NS * K)
    srcp = jnp.concatenate(
        [src, jnp.zeros((epa - E,), jnp.int32)]).reshape(NS, nba, K)
    sidx = jnp.stack([srcp, srcp + NT])
    didx = jnp.concatenate(
        [dst, jnp.full((epa - E,), N, jnp.int32)]).reshape(NS, nba, K)

    zeros16 = jnp.zeros((NT, HHALF), jnp.float32)
    ones16 = jnp.ones((16, HHALF), jnp.float32)

    deg2 = _make_deg_call(nbd)(didx_deg, zeros16, ones16)
    hs1, dinv = _prep_call(x, W1, deg2)
    agg = _make_agg_call()
    S1 = agg(hs1.reshape(NC * NT, HHALF), sidx, didx)
    hs2 = _mid_call(S1, dinv, W2, b1.reshape(1, H), g1.reshape(1, H),
                    be1.reshape(1, H), rm1.reshape(1, H), rv1.reshape(1, H))
    S2 = agg(hs2.reshape(NC * NT, HHALF), sidx, didx)
    out = _fin_call(S2, dinv, W3, b2.reshape(1, H), g2.reshape(1, H),
                    be2.reshape(1, H), rm2.reshape(1, H), rv2.reshape(1, H),
                    b3.reshape(1, C))
    return out


# speedup vs baseline: 10.5095x; 1.3634x over previous
import functools

import jax
import jax.numpy as jnp
from jax import lax
from jax.experimental import pallas as pl
from jax.experimental.pallas import tpu as pltpu
from jax.experimental.pallas import tpu_sc as plsc

N = 10000
F_IN = 128
H = 256
C = 40
HHALF = H // 2

NC = 2
NS = 16
K = 128

BT = 128
NT = 79 * BT
RPT = NT // NS

_MESH = plsc.VectorSubcoreMesh(
    core_axis_name="c", subcore_axis_name="s", num_cores=NC, num_subcores=NS)


def _deg_body(didx, zeros, ones, out, ones_v, di0, di1, is0, is1, ss, so,
              acc):
    c = lax.axis_index("c")
    s = lax.axis_index("s")
    wid = c * NS + s
    nb = didx.shape[1]
    pltpu.async_copy(ones, ones_v, so).wait()
    pltpu.async_copy(zeros.at[pl.ds(s * RPT, RPT)],
                     acc.at[pl.ds(s * RPT, RPT)], so).wait()
    pltpu.async_copy(didx.at[wid, 0], di0, is0)
    pltpu.async_copy(didx.at[wid, 1], di1, is1)
    plsc.subcore_barrier()

    def pair(b2, carry):
        b = 2 * b2
        for off, di, isem in ((0, di0, is0), (1, di1, is1)):
            pltpu.make_async_copy(didx.at[wid, 0], di, isem).wait()
            for j in range(K // 16):
                iv = di[pl.ds(j * 16, 16)]
                pltpu.async_copy(ones_v, acc.at[iv], ss, add=True)

            @pl.when(b + off + 2 < nb)
            def _():
                pltpu.async_copy(didx.at[wid, b + off + 2], di, isem)

        return carry

    lax.fori_loop(0, nb // 2, pair, 0)
    iv0 = lax.iota(jnp.int32, 16)

    def drain(i, carry):
        pltpu.make_async_copy(ones_v, acc.at[iv0], ss).wait()
        return carry

    lax.fori_loop(0, nb * (K // 16), drain, 0)
    plsc.subcore_barrier()
    pltpu.async_copy(acc.at[pl.ds(s * RPT, RPT)],
                     out.at[c, pl.ds(s * RPT, RPT)], so).wait()


def _agg_body(table, sidx, didx, out, si0, si1, di0, di1, b0, b1, is0, is1,
              gs0, gs1, ss0, ss1, so, acc):
    c = lax.axis_index("c")
    s = lax.axis_index("s")
    nb = sidx.shape[2]
    pltpu.async_copy(table.at[pl.ds(c * NT + s * RPT, RPT)],
                     acc.at[pl.ds(s * RPT, RPT)], so).wait()
    for off, si, di, isem in ((0, si0, di0, is0), (1, si1, di1, is1)):
        pltpu.async_copy(sidx.at[c, s, off], si, isem)
        pltpu.async_copy(didx.at[s, off], di, isem)
        pltpu.make_async_copy(sidx.at[c, s, off], si, isem).wait()
        pltpu.make_async_copy(didx.at[s, off], di, isem).wait()
    plsc.subcore_barrier()
    pltpu.async_copy(table.at[si0], b0, gs0)
    pltpu.async_copy(table.at[si1], b1, gs1)

    def pair(b2, carry):
        b = 2 * b2
        for off, si, di, bf, isem, gsem, ssem in (
            (0, si0, di0, b0, is0, gs0, ss0),
            (1, si1, di1, b1, is1, gs1, ss1),
        ):
            pltpu.make_async_copy(table.at[si], bf, gsem).wait()
            for j in range(K // 16):
                iv = di[pl.ds(j * 16, 16)]
                pltpu.async_copy(bf.at[pl.ds(j * 16, 16)], acc.at[iv], ssem,
                                 add=True)

            @pl.when(b + off + 2 < nb)
            def _():
                pltpu.async_copy(sidx.at[c, s, b + off + 2], si, isem)
                pltpu.async_copy(didx.at[s, b + off + 2], di, isem)

        iv0 = lax.iota(jnp.int32, 16)
        for off, si, di, bf, isem, gsem, ssem in (
            (0, si0, di0, b0, is0, gs0, ss0),
            (1, si1, di1, b1, is1, gs1, ss1),
        ):
            for j in range(K // 16):
                pltpu.make_async_copy(bf.at[pl.ds(j * 16, 16)], acc.at[iv0],
                                      ssem).wait()

            @pl.when(b + off + 2 < nb)
            def _():
                pltpu.make_async_copy(sidx.at[c, s, 0], si, isem).wait()
                pltpu.make_async_copy(didx.at[s, 0], di, isem).wait()
                pltpu.async_copy(table.at[si], bf, gsem)

        return carry

    lax.fori_loop(0, nb // 2, pair, 0)
    plsc.subcore_barrier()
    pltpu.async_copy(acc.at[pl.ds(s * RPT, RPT)],
                     out.at[c, pl.ds(s * RPT, RPT)], so).wait()


def _make_deg_call(nbd):
    return pl.kernel(
        _deg_body,
        out_type=jax.ShapeDtypeStruct((NC, NT, HHALF), jnp.float32),
        mesh=_MESH,
        scratch_types=[
            pltpu.VMEM((16, HHALF), jnp.float32),
            pltpu.VMEM((K,), jnp.int32),
            pltpu.VMEM((K,), jnp.int32),
            pltpu.SemaphoreType.DMA,
            pltpu.SemaphoreType.DMA,
            pltpu.SemaphoreType.DMA,
            pltpu.SemaphoreType.DMA,
            pltpu.VMEM_SHARED((NT, HHALF), jnp.float32),
        ],
    )


def _make_agg_call():
    return pl.kernel(
        _agg_body,
        out_type=jax.ShapeDtypeStruct((NC, NT, HHALF), jnp.float32),
        mesh=_MESH,
        scratch_types=[
            pltpu.VMEM((K,), jnp.int32),
            pltpu.VMEM((K,), jnp.int32),
            pltpu.VMEM((K,), jnp.int32),
            pltpu.VMEM((K,), jnp.int32),
            pltpu.VMEM((K, HHALF), jnp.float32),
            pltpu.VMEM((K, HHALF), jnp.float32),
            pltpu.SemaphoreType.DMA,
            pltpu.SemaphoreType.DMA,
            pltpu.SemaphoreType.DMA,
            pltpu.SemaphoreType.DMA,
            pltpu.SemaphoreType.DMA,
            pltpu.SemaphoreType.DMA,
            pltpu.SemaphoreType.DMA,
            pltpu.VMEM_SHARED((NT, HHALF), jnp.float32),
        ],
    )


def _prep_body(x_ref, w1_ref, deg_ref, hs_ref, dinv_ref):
    deg = deg_ref[0] + deg_ref[1]
    dinv = lax.rsqrt(deg[:, :1] + 1.0)
    h = jnp.dot(x_ref[...], w1_ref[...], preferred_element_type=jnp.float32)
    hs = h * dinv
    hs_ref[0] = hs[:, :HHALF]
    hs_ref[1] = hs[:, HHALF:]
    dinv_ref[...] = dinv * jnp.ones((1, 16), jnp.float32)


def _bn_relu(S_ref, dinv_ref, b_ref, g_ref, be_ref, rm_ref, rv_ref):
    dinv = dinv_ref[:, :1]
    agg = jnp.concatenate([S_ref[0], S_ref[1]], axis=1) * dinv + b_ref[...]
    scale = g_ref[...] * lax.rsqrt(rv_ref[...] + 1e-5)
    v = (agg - rm_ref[...]) * scale + be_ref[...]
    return jnp.maximum(v, 0.0), dinv


def _mid_body(S_ref, dinv_ref, w2_ref, b_ref, g_ref, be_ref, rm_ref, rv_ref,
              hs_ref):
    v, dinv = _bn_relu(S_ref, dinv_ref, b_ref, g_ref, be_ref, rm_ref, rv_ref)
    h2 = jnp.dot(v, w2_ref[...], preferred_element_type=jnp.float32)
    hs = h2 * dinv
    hs_ref[0] = hs[:, :HHALF]
    hs_ref[1] = hs[:, HHALF:]


def _fin_body(S_ref, dinv_ref, w3_ref, b_ref, g_ref, be_ref, rm_ref, rv_ref,
              b3_ref, out_ref):
    v, _ = _bn_relu(S_ref, dinv_ref, b_ref, g_ref, be_ref, rm_ref, rv_ref)
    out_ref[...] = (jnp.dot(v, w3_ref[...], preferred_element_type=jnp.float32)
                    + b3_ref[...])


_GRID = NT // BT

_row_spec = pl.BlockSpec((BT, F_IN), lambda i: (i, 0))
_S_spec = pl.BlockSpec((NC, BT, HHALF), lambda i: (0, i, 0))
_deg_spec = pl.BlockSpec((NC, BT, HHALF), lambda i: (0, i, 0))
_dinv_spec = pl.BlockSpec((BT, 16), lambda i: (i, 0))
_vecH_spec = pl.BlockSpec((1, H), lambda i: (0, 0))


def _prep_call(x, W1, deg2):
    return pl.pallas_call(
        _prep_body,
        grid=(_GRID,),
        in_specs=[_row_spec,
                  pl.BlockSpec((F_IN, H), lambda i: (0, 0)),
                  _deg_spec],
        out_specs=[pl.BlockSpec((NC, BT, HHALF), lambda i: (0, i, 0)),
                   _dinv_spec],
        out_shape=[jax.ShapeDtypeStruct((NC, NT, HHALF), jnp.float32),
                   jax.ShapeDtypeStruct((NT, 16), jnp.float32)],
    )(x, W1, deg2)


def _mid_call(S1, dinv, W2, b, g, be, rm, rv):
    return pl.pallas_call(
        _mid_body,
        grid=(_GRID,),
        in_specs=[_S_spec, _dinv_spec,
                  pl.BlockSpec((H, H), lambda i: (0, 0)),
                  _vecH_spec, _vecH_spec, _vecH_spec, _vecH_spec, _vecH_spec],
        out_specs=pl.BlockSpec((NC, BT, HHALF), lambda i: (0, i, 0)),
        out_shape=jax.ShapeDtypeStruct((NC, NT, HHALF), jnp.float32),
    )(S1, dinv, W2, b, g, be, rm, rv)


def _fin_call(S2, dinv, W3, b, g, be, rm, rv, b3):
    return pl.pallas_call(
        _fin_body,
        grid=(_GRID,),
        in_specs=[_S_spec, _dinv_spec,
                  pl.BlockSpec((H, C), lambda i: (0, 0)),
                  _vecH_spec, _vecH_spec, _vecH_spec, _vecH_spec, _vecH_spec,
                  pl.BlockSpec((1, C), lambda i: (0, 0))],
        out_specs=pl.BlockSpec((BT, C), lambda i: (i, 0)),
        out_shape=jax.ShapeDtypeStruct((N, C), jnp.float32),
    )(S2, dinv, W3, b, g, be, rm, rv, b3)


def _ceil_to(a, m):
    return -(-a // m) * m


def kernel(x, edge_index, W1, b1, g1, be1, rm1, rv1, W2, b2, g2, be2, rm2,
           rv2, W3, b3):
    src = edge_index[0].astype(jnp.int32)
    dst = edge_index[1].astype(jnp.int32)
    E = src.shape[0]

    epd = _ceil_to(E, NC * NS * K * 2)
    nbd = epd // (NC * NS * K)
    didx_deg = jnp.concatenate(
        [dst, jnp.full((epd - E,), N, jnp.int32)]).reshape(NC * NS, nbd, K)

    epa = _ceil_to(E, NS * K * 2)
    nba = epa // (NS * K)
    srcp = jnp.concatenate(
        [src, jnp.zeros((epa - E,), jnp.int32)]).reshape(NS, nba, K)
    sidx = jnp.stack([srcp, srcp + NT])
    didx = jnp.concatenate(
        [dst, jnp.full((epa - E,), N, jnp.int32)]).reshape(NS, nba, K)

    zeros16 = jnp.zeros((NT, HHALF), jnp.float32)
    ones16 = jnp.ones((16, HHALF), jnp.float32)

    deg2 = _make_deg_call(nbd)(didx_deg, zeros16, ones16)
    hs1, dinv = _prep_call(x, W1, deg2)
    agg = _make_agg_call()
    S1 = agg(hs1.reshape(NC * NT, HHALF), sidx, didx)
    hs2 = _mid_call(S1, dinv, W2, b1.reshape(1, H), g1.reshape(1, H),
                    be1.reshape(1, H), rm1.reshape(1, H), rv1.reshape(1, H))
    S2 = agg(hs2.reshape(NC * NT, HHALF), sidx, didx)
    out = _fin_call(S2, dinv, W3, b2.reshape(1, H), g2.reshape(1, H),
                    be2.reshape(1, H), rm2.reshape(1, H), rv2.reshape(1, H),
                    b3.reshape(1, C))
    return out
